# baseline (speedup 1.0000x reference)
import jax
import jax.numpy as jnp
from jax.experimental import pallas as pl
from jax.experimental.pallas import tpu as pltpu

_N = 10000
_BI = 200
_NI = _N // _BI
_HIGH = jax.lax.Precision.HIGHEST


def _mm(a, b, precision):
    return jax.lax.dot_general(a, b, (((1,), (0,)), ((), ())),
                               precision=precision,
                               preferred_element_type=jnp.float32)


def _lin_kernel(x_ref, w_ref, out_ref):
    out_ref[...] = _mm(x_ref[...], w_ref[...], _HIGH)


def _lin(x, w):
    f_in, f_out = w.shape
    return pl.pallas_call(
        _lin_kernel,
        grid=(_NI,),
        in_specs=[pl.BlockSpec((_BI, f_in), lambda i: (i, 0)),
                  pl.BlockSpec((f_in, f_out), lambda i: (0, 0))],
        out_specs=pl.BlockSpec((_BI, f_out), lambda i: (i, 0)),
        out_shape=jax.ShapeDtypeStruct((_N, f_out), jnp.float32),
        compiler_params=pltpu.CompilerParams(
            dimension_semantics=("parallel",)),
    )(x, w)


def _hop_kernel(a_ref, s_ref, out_ref):
    out_ref[...] = _mm(a_ref[...], s_ref[...], _HIGH)


def _hop(adj, s):
    f = s.shape[1]
    return pl.pallas_call(
        _hop_kernel,
        grid=(_NI,),
        in_specs=[pl.BlockSpec((_BI, _N), lambda i: (i, 0)),
                  pl.BlockSpec((_N, f), lambda i: (0, 0))],
        out_specs=pl.BlockSpec((_BI, f), lambda i: (i, 0)),
        out_shape=jax.ShapeDtypeStruct((_N, f), jnp.float32),
        compiler_params=pltpu.CompilerParams(
            dimension_semantics=("parallel",)),
    )(adj, s)


def _hop3a_kernel(a_ref, s2_ref, s0_ref, s1_ref, b0_ref, w1_ref, out_ref):
    i = pl.program_id(0)
    s3 = _mm(a_ref[...], s2_ref[...], _HIGH)
    s2_tile = s2_ref[pl.ds(i * _BI, _BI), :]
    h = s0_ref[...] + s1_ref[...] + s2_tile + s3 + b0_ref[...]
    h = jnp.maximum(h, 0.0)
    out_ref[...] = _mm(h, w1_ref[...], _HIGH)


def _hop3a(adj, s2, s0, s1, b0, w1):
    f = s0.shape[1]
    f_out = w1.shape[1]
    return pl.pallas_call(
        _hop3a_kernel,
        grid=(_NI,),
        in_specs=[pl.BlockSpec((_BI, _N), lambda i: (i, 0)),
                  pl.BlockSpec((_N, f), lambda i: (0, 0)),
                  pl.BlockSpec((_BI, f), lambda i: (i, 0)),
                  pl.BlockSpec((_BI, f), lambda i: (i, 0)),
                  pl.BlockSpec((1, f), lambda i: (0, 0)),
                  pl.BlockSpec((f, f_out), lambda i: (0, 0))],
        out_specs=pl.BlockSpec((_BI, f_out), lambda i: (i, 0)),
        out_shape=jax.ShapeDtypeStruct((_N, f_out), jnp.float32),
        compiler_params=pltpu.CompilerParams(
            dimension_semantics=("parallel",)),
    )(adj, s2, s0, s1, b0, w1)


def _hop3b_kernel(a_ref, t2_ref, t0_ref, t1_ref, b1_ref, out_ref):
    i = pl.program_id(0)
    t3 = _mm(a_ref[...], t2_ref[...], _HIGH)
    t2_tile = t2_ref[pl.ds(i * _BI, _BI), :]
    logits = t0_ref[...] + t1_ref[...] + t2_tile + t3 + b1_ref[...]
    m = jnp.max(logits, axis=1, keepdims=True)
    lse = m + jnp.log(jnp.sum(jnp.exp(logits - m), axis=1, keepdims=True))
    out_ref[...] = logits - lse


def _hop3b(adj, t2, t0, t1, b1):
    f = t0.shape[1]
    return pl.pallas_call(
        _hop3b_kernel,
        grid=(_NI,),
        in_specs=[pl.BlockSpec((_BI, _N), lambda i: (i, 0)),
                  pl.BlockSpec((_N, f), lambda i: (0, 0)),
                  pl.BlockSpec((_BI, f), lambda i: (i, 0)),
                  pl.BlockSpec((_BI, f), lambda i: (i, 0)),
                  pl.BlockSpec((1, f), lambda i: (0, 0))],
        out_specs=pl.BlockSpec((_BI, f), lambda i: (i, 0)),
        out_shape=jax.ShapeDtypeStruct((_N, f), jnp.float32),
        compiler_params=pltpu.CompilerParams(
            dimension_semantics=("parallel",)),
    )(adj, t2, t0, t1, b1)


def kernel(x, adj, W0, b0, W1, b1):
    b0r = b0.reshape(1, -1)
    b1r = b1.reshape(1, -1)
    s0 = _lin(x, W0)
    s1 = _hop(adj, s0)
    s2 = _hop(adj, s1)
    t0 = _hop3a(adj, s2, s0, s1, b0r, W1)
    t1 = _hop(adj, t0)
    t2 = _hop(adj, t1)
    return _hop3b(adj, t2, t0, t1, b1r)


# speedup vs baseline: 3.1494x; 3.1494x over previous
import jax
import jax.numpy as jnp
from jax.experimental import pallas as pl
from jax.experimental.pallas import tpu as pltpu

_N = 10000
_BI = 200
_NI = _N // _BI
_HIGH = jax.lax.Precision.HIGHEST
_BF16 = jnp.bfloat16


def _mm(a, b, precision=None):
    return jax.lax.dot_general(a, b, (((1,), (0,)), ((), ())),
                               precision=precision,
                               preferred_element_type=jnp.float32)


def _parallel(n):
    return pltpu.CompilerParams(dimension_semantics=("parallel",) * n)


def _lin_kernel(x_ref, w_ref, out_ref, outb_ref):
    s0 = _mm(x_ref[...], w_ref[...], _HIGH)
    out_ref[...] = s0
    outb_ref[...] = s0.astype(_BF16)


def _lin(x, w):
    f_in, f_out = w.shape
    return pl.pallas_call(
        _lin_kernel,
        grid=(_NI,),
        in_specs=[pl.BlockSpec((_BI, f_in), lambda i: (i, 0)),
                  pl.BlockSpec((f_in, f_out), lambda i: (0, 0))],
        out_specs=[pl.BlockSpec((_BI, f_out), lambda i: (i, 0)),
                   pl.BlockSpec((_BI, f_out), lambda i: (i, 0))],
        out_shape=[jax.ShapeDtypeStruct((_N, f_out), jnp.float32),
                   jax.ShapeDtypeStruct((_N, f_out), _BF16)],
        compiler_params=_parallel(1),
    )(x, w)


def _hop1_kernel(a_ref, s_ref, out_ref, ab_ref):
    ab = a_ref[...].astype(_BF16)
    ab_ref[...] = ab
    out_ref[...] = _mm(ab, s_ref[...]).astype(_BF16)


def _hop1(adj, s):
    f = s.shape[1]
    return pl.pallas_call(
        _hop1_kernel,
        grid=(_NI,),
        in_specs=[pl.BlockSpec((_BI, _N), lambda i: (i, 0)),
                  pl.BlockSpec((_N, f), lambda i: (0, 0))],
        out_specs=[pl.BlockSpec((_BI, f), lambda i: (i, 0)),
                   pl.BlockSpec((_BI, _N), lambda i: (i, 0))],
        out_shape=[jax.ShapeDtypeStruct((_N, f), _BF16),
                   jax.ShapeDtypeStruct((_N, _N), _BF16)],
        compiler_params=_parallel(1),
    )(adj, s)


def _hop_kernel(a_ref, s_ref, out_ref):
    out_ref[...] = _mm(a_ref[...], s_ref[...]).astype(_BF16)


def _hop(adj_b, s):
    f = s.shape[1]
    return pl.pallas_call(
        _hop_kernel,
        grid=(_NI,),
        in_specs=[pl.BlockSpec((_BI, _N), lambda i: (i, 0)),
                  pl.BlockSpec((_N, f), lambda i: (0, 0))],
        out_specs=pl.BlockSpec((_BI, f), lambda i: (i, 0)),
        out_shape=jax.ShapeDtypeStruct((_N, f), _BF16),
        compiler_params=_parallel(1),
    )(adj_b, s)


def _hop3a_kernel(a_ref, s2_ref, s0_ref, s1_ref, b0_ref, w1_ref,
                  out_ref, outb_ref):
    i = pl.program_id(0)
    s3 = _mm(a_ref[...], s2_ref[...])
    s2_tile = s2_ref[pl.ds(i * _BI, _BI), :].astype(jnp.float32)
    h = (s0_ref[...] + s1_ref[...].astype(jnp.float32) + s2_tile + s3
         + b0_ref[...])
    h = jnp.maximum(h, 0.0)
    t0 = _mm(h, w1_ref[...], _HIGH)
    out_ref[...] = t0
    outb_ref[...] = t0.astype(_BF16)


def _hop3a(adj_b, s2, s0, s1, b0, w1):
    f = s0.shape[1]
    f_out = w1.shape[1]
    return pl.pallas_call(
        _hop3a_kernel,
        grid=(_NI,),
        in_specs=[pl.BlockSpec((_BI, _N), lambda i: (i, 0)),
                  pl.BlockSpec((_N, f), lambda i: (0, 0)),
                  pl.BlockSpec((_BI, f), lambda i: (i, 0)),
                  pl.BlockSpec((_BI, f), lambda i: (i, 0)),
                  pl.BlockSpec((1, f), lambda i: (0, 0)),
                  pl.BlockSpec((f, f_out), lambda i: (0, 0))],
        out_specs=[pl.BlockSpec((_BI, f_out), lambda i: (i, 0)),
                   pl.BlockSpec((_BI, f_out), lambda i: (i, 0))],
        out_shape=[jax.ShapeDtypeStruct((_N, f_out), jnp.float32),
                   jax.ShapeDtypeStruct((_N, f_out), _BF16)],
        compiler_params=_parallel(1),
    )(adj_b, s2, s0, s1, b0, w1)


def _hop3b_kernel(a_ref, t2_ref, t0_ref, t1_ref, b1_ref, out_ref):
    i = pl.program_id(0)
    t3 = _mm(a_ref[...], t2_ref[...])
    t2_tile = t2_ref[pl.ds(i * _BI, _BI), :].astype(jnp.float32)
    logits = (t0_ref[...] + t1_ref[...].astype(jnp.float32) + t2_tile + t3
              + b1_ref[...])
    m = jnp.max(logits, axis=1, keepdims=True)
    lse = m + jnp.log(jnp.sum(jnp.exp(logits - m), axis=1, keepdims=True))
    out_ref[...] = logits - lse


def _hop3b(adj_b, t2, t0, t1, b1):
    f = t0.shape[1]
    return pl.pallas_call(
        _hop3b_kernel,
        grid=(_NI,),
        in_specs=[pl.BlockSpec((_BI, _N), lambda i: (i, 0)),
                  pl.BlockSpec((_N, f), lambda i: (0, 0)),
                  pl.BlockSpec((_BI, f), lambda i: (i, 0)),
                  pl.BlockSpec((_BI, f), lambda i: (i, 0)),
                  pl.BlockSpec((1, f), lambda i: (0, 0))],
        out_specs=pl.BlockSpec((_BI, f), lambda i: (i, 0)),
        out_shape=jax.ShapeDtypeStruct((_N, f), jnp.float32),
        compiler_params=_parallel(1),
    )(adj_b, t2, t0, t1, b1)


def kernel(x, adj, W0, b0, W1, b1):
    b0r = b0.reshape(1, -1)
    b1r = b1.reshape(1, -1)
    s0, s0b = _lin(x, W0)
    s1, adj_b = _hop1(adj, s0b)
    s2 = _hop(adj_b, s1)
    t0, t0b = _hop3a(adj_b, s2, s0, s1, b0r, W1)
    t1 = _hop(adj_b, t0b)
    t2 = _hop(adj_b, t1)
    return _hop3b(adj_b, t2, t0, t1, b1r)
